# baseline (speedup 1.0000x reference)
import jax
import jax.numpy as jnp
from jax import lax
from jax.experimental import pallas as pl
from jax.experimental.pallas import tpu as pltpu

_BB = 2048


def _mlp_body(seqs_ref, q_ref, et_ref, w1at_ref, w1bt_ref, b1_ref, w2_ref, b2_ref, out_ref):
    mem = seqs_ref[:, 15:23]
    qt = q_ref[...]
    toks = jnp.concatenate([qt, mem], axis=1)
    tokst = jnp.transpose(toks).astype(jnp.bfloat16)

    bb = tokst.shape[1]
    iota = lax.broadcasted_iota(jnp.int32, (64, bb), 0).astype(jnp.bfloat16)

    one = jnp.bfloat16(1.0)
    zero = jnp.bfloat16(0.0)
    ohq = jnp.where(tokst[0:1, :] == iota, one, zero)
    cnt = jnp.where(tokst[1:2, :] == iota, one, zero)
    for t in range(2, 9):
        cnt = cnt + jnp.where(tokst[t : t + 1, :] == iota, one, zero)

    et = et_ref[...]
    ones_row = jnp.full((1, 64), 1.0, dtype=jnp.float32)
    m1t = (
        jnp.dot(w1at_ref[...], et, preferred_element_type=jnp.float32)
        + jnp.dot(b1_ref[...], ones_row, preferred_element_type=jnp.float32)
    )
    m2t = jnp.dot(w1bt_ref[...], et, preferred_element_type=jnp.float32) * 0.125

    preact_t = jnp.dot(
        m1t.astype(jnp.bfloat16), ohq, preferred_element_type=jnp.float32
    ) + jnp.dot(
        m2t.astype(jnp.bfloat16), cnt, preferred_element_type=jnp.float32
    )
    h1t = jnp.maximum(preact_t, 0.0)

    out = lax.dot_general(
        h1t,
        w2_ref[...],
        dimension_numbers=(((0,), (0,)), ((), ())),
        preferred_element_type=jnp.float32,
    )
    out_ref[...] = out + b2_ref[...]


def kernel(seqs, query_tok, embed, W1, b1, W2, b2):
    B = seqs.shape[0]
    et = embed[:64].T
    w1at = W1[:64].T
    w1bt = W1[64:].T

    grid = (B // _BB,)
    return pl.pallas_call(
        _mlp_body,
        grid=grid,
        in_specs=[
            pl.BlockSpec((_BB, 24), lambda i: (i, 0)),
            pl.BlockSpec((_BB, 1), lambda i: (i, 0)),
            pl.BlockSpec((64, 64), lambda i: (0, 0)),
            pl.BlockSpec((64, 64), lambda i: (0, 0)),
            pl.BlockSpec((64, 64), lambda i: (0, 0)),
            pl.BlockSpec((64, 1), lambda i: (0, 0)),
            pl.BlockSpec((64, 64), lambda i: (0, 0)),
            pl.BlockSpec((1, 64), lambda i: (0, 0)),
        ],
        out_specs=pl.BlockSpec((_BB, 64), lambda i: (i, 0)),
        out_shape=jax.ShapeDtypeStruct((B, 64), jnp.float32),
    )(
        seqs.astype(jnp.int32),
        query_tok.astype(jnp.int32)[:, None],
        et,
        w1at,
        w1bt,
        b1[:, None],
        W2,
        b2[None, :],
    )
